# baseline (speedup 1.0000x reference)
import functools

import jax
import jax.numpy as jnp
from jax import lax
from jax.experimental import pallas as pl
from jax.experimental.pallas import tpu as pltpu
from jax.experimental.pallas import tpu_sc as plsc

_K = 8
_BETA = 0.5
_EPS = 1e-8


def _tc_body(c_blk_ref, s_full_ref, t_full_ref, c_full_ref, pk_ref, idx_ref,
             s_scr, t_scr):
    b = pl.program_id(0)
    rb = pl.program_id(1)
    R = c_blk_ref.shape[1]
    P = s_full_ref.shape[1]
    D = s_full_ref.shape[2]
    ones_d = jnp.ones((1, D), jnp.float32)
    nt_dims = (((1,), (1,)), ((), ()))
    hi_mask = jnp.uint32(0xFFFF0000)

    @pl.when(rb == 0)
    def _normalize_full():
        for full_ref, scr in ((s_full_ref, s_scr), (t_full_ref, t_scr)):
            x = full_ref[0]
            n2 = lax.dot_general(x * x, ones_d, nt_dims,
                                 preferred_element_type=jnp.float32)
            inv = 1.0 / jnp.maximum(jnp.sqrt(n2), _EPS)
            scr[...] = (x * inv).astype(jnp.bfloat16)

    s_blk = s_scr[pl.ds(rb * R, R), :]
    t_blk = t_scr[pl.ds(rb * R, R), :]
    gs = lax.dot_general(s_blk, s_scr[...], nt_dims,
                         preferred_element_type=jnp.float32)
    gt = lax.dot_general(t_blk, t_scr[...], nt_dims,
                         preferred_element_type=jnp.float32)

    rnd = jnp.uint32(0x8000)
    pk_hi = (lax.bitcast_convert_type(gs, jnp.uint32) + rnd) & hi_mask
    pk_lo = (lax.bitcast_convert_type(gt, jnp.uint32) + rnd) >> 16
    pk_ref[0] = lax.bitcast_convert_type(pk_hi | pk_lo, jnp.int32)

    c_blk = c_blk_ref[0]
    c_full = c_full_ref[0]
    ones_3 = jnp.ones((1, 3), jnp.float32)
    dotc = lax.dot_general(c_blk, c_full, nt_dims,
                           preferred_element_type=jnp.float32)
    n2c_blk = jnp.sum(c_blk * c_blk, axis=1, keepdims=True)
    n2c_full = lax.dot_general(ones_3, c_full * c_full, nt_dims,
                               preferred_element_type=jnp.float32)
    d2 = jnp.maximum(n2c_blk + n2c_full - 2.0 * dotc, 0.0)

    col = lax.broadcasted_iota(jnp.int32, (R, P), 1)
    row = lax.broadcasted_iota(jnp.int32, (R, P), 0) + rb * R
    d2 = jnp.where(col == row, jnp.float32(jnp.inf), d2)

    ck = lax.bitcast_convert_type(
        (lax.bitcast_convert_type(d2, jnp.uint32) & jnp.uint32(0xFFFFFC00))
        | lax.bitcast_convert_type(col, jnp.uint32).astype(jnp.uint32),
        jnp.int32)

    imax = jnp.int32(2**31 - 1)
    cols = []
    for _ in range(_K):
        m = jnp.min(ck, axis=1, keepdims=True)
        ck = jnp.where(ck == m, imax, ck)
        cols.append(m & jnp.int32(P - 1))
    r_base = (b * P + rb * R
              + lax.broadcasted_iota(jnp.int32, (R, 1), 0)) * P
    idx_ref[0] = jnp.concatenate(cols, axis=1) + r_base


def _sc_body(pk_hbm, idx_hbm, out_hbm, idx_v, vals_v, acc_v, sem, *,
             n_dmas, n_chunks):
    wid = lax.axis_index("s") * 2 + lax.axis_index("c")
    pltpu.sync_copy(idx_hbm.at[wid], idx_v)
    copies = [
        pltpu.async_copy(pk_hbm.at[idx_v.at[j]],
                         vals_v.at[pl.ds(j * 128, 128)], sem)
        for j in range(n_dmas)
    ]
    for c in copies:
        c.wait()

    hi = jnp.uint32(0xFFFF0000)

    def chunk(i, acc):
        v = vals_v[pl.ds(i * 16, 16)]
        vu = lax.bitcast_convert_type(v, jnp.uint32)
        cs = lax.bitcast_convert_type(vu & hi, jnp.float32)
        ct = lax.bitcast_convert_type(vu << 16, jnp.float32)
        ax = jnp.abs(cs - ct)
        return acc + jnp.where(ax < _BETA, 0.5 * ax * ax / _BETA,
                               ax - 0.5 * _BETA)

    acc_v[...] = lax.fori_loop(0, n_chunks, chunk,
                               jnp.zeros((16,), jnp.float32))
    pltpu.sync_copy(acc_v, out_hbm.at[wid])


def kernel(student_emb, teacher_emb, centers):
    B, P, D = student_emb.shape
    R = min(256, P)
    nrb = P // R
    gb = 2 if B % 2 == 0 else B
    n_groups = B // gb

    cblk = pl.BlockSpec((1, R, 3), lambda b, rb: (b, rb if R != P else 0, 0))
    full = pl.BlockSpec((1, P, D), lambda b, rb: (b, 0, 0))
    cfull = pl.BlockSpec((1, P, 3), lambda b, rb: (b, 0, 0))

    tc = pl.pallas_call(
        _tc_body,
        grid=(gb, nrb),
        in_specs=[cblk, full, full, cfull],
        out_specs=[pl.BlockSpec((1, R, P), lambda b, rb: (b, rb, 0)),
                   pl.BlockSpec((1, R, _K), lambda b, rb: (b, rb, 0))],
        out_shape=[jax.ShapeDtypeStruct((gb, P, P), jnp.int32),
                   jax.ShapeDtypeStruct((gb, P, _K), jnp.int32)],
        scratch_shapes=[pltpu.VMEM((P, D), jnp.bfloat16),
                        pltpu.VMEM((P, D), jnp.bfloat16)],
        compiler_params=pltpu.CompilerParams(
            dimension_semantics=("arbitrary", "arbitrary")),
    )

    info = plsc.get_sparse_core_info()
    nw = info.num_cores * info.num_subcores
    n_pairs = gb * P * _K
    per_w = n_pairs // nw
    n_dmas = per_w // 128
    n_chunks = per_w // 16

    sc = functools.partial(
        pl.kernel,
        mesh=plsc.VectorSubcoreMesh(core_axis_name="c", subcore_axis_name="s"),
        out_type=jax.ShapeDtypeStruct((nw, 16), jnp.float32),
        scratch_types=[pltpu.VMEM((n_dmas, 128), jnp.int32),
                       pltpu.VMEM((per_w,), jnp.int32),
                       pltpu.VMEM((16,), jnp.float32),
                       pltpu.SemaphoreType.DMA],
    )(functools.partial(_sc_body, n_dmas=n_dmas, n_chunks=n_chunks))

    partials = []
    for g in range(n_groups):
        sl = slice(g * gb, (g + 1) * gb)
        pk, idx = tc(centers[sl], student_emb[sl], teacher_emb[sl],
                     centers[sl])
        partials.append(sc(pk.reshape(-1), idx.reshape(nw, n_dmas, 128)))
    return jnp.sum(jnp.stack(partials)) / jnp.float32(B * P * _K)


# speedup vs baseline: 1.1900x; 1.1900x over previous
import functools

import jax
import jax.numpy as jnp
from jax import lax
from jax.experimental import pallas as pl
from jax.experimental.pallas import tpu as pltpu
from jax.experimental.pallas import tpu_sc as plsc

_K = 8
_BETA = 0.5
_EPS = 1e-8


def _norm_body(s_ref, t_ref, sn_ref, tn_ref):
    D = s_ref.shape[2]
    ones_d = jnp.ones((1, D), jnp.float32)
    nt_dims = (((1,), (1,)), ((), ()))
    for src, dst in ((s_ref, sn_ref), (t_ref, tn_ref)):
        x = src[0]
        n2 = lax.dot_general(x * x, ones_d, nt_dims,
                             preferred_element_type=jnp.float32)
        inv = 1.0 / jnp.maximum(jnp.sqrt(n2), _EPS)
        dst[0] = (x * inv).astype(jnp.bfloat16)


def _tc_body(c_blk_ref, s_blk_ref, t_blk_ref, s_full_ref, t_full_ref,
             c_full_ref, pk_ref, idx_ref):
    b = pl.program_id(0)
    rb = pl.program_id(1)
    R = c_blk_ref.shape[1]
    P = s_full_ref.shape[1]
    nt_dims = (((1,), (1,)), ((), ()))
    hi_mask = jnp.uint32(0xFFFF0000)

    gs = lax.dot_general(s_blk_ref[0], s_full_ref[0], nt_dims,
                         preferred_element_type=jnp.float32)
    gt = lax.dot_general(t_blk_ref[0], t_full_ref[0], nt_dims,
                         preferred_element_type=jnp.float32)

    rnd = jnp.uint32(0x8000)
    pk_hi = (lax.bitcast_convert_type(gs, jnp.uint32) + rnd) & hi_mask
    pk_lo = (lax.bitcast_convert_type(gt, jnp.uint32) + rnd) >> 16
    pk_ref[0] = lax.bitcast_convert_type(pk_hi | pk_lo, jnp.int32)

    c_blk = c_blk_ref[0]
    c_full = c_full_ref[0]
    ones_3 = jnp.ones((1, 3), jnp.float32)
    dotc = lax.dot_general(c_blk, c_full, nt_dims,
                           preferred_element_type=jnp.float32)
    n2c_blk = jnp.sum(c_blk * c_blk, axis=1, keepdims=True)
    n2c_full = lax.dot_general(ones_3, c_full * c_full, nt_dims,
                               preferred_element_type=jnp.float32)
    d2 = jnp.maximum(n2c_blk + n2c_full - 2.0 * dotc, 0.0)

    col = lax.broadcasted_iota(jnp.int32, (R, P), 1)
    row = lax.broadcasted_iota(jnp.int32, (R, P), 0) + rb * R
    d2 = jnp.where(col == row, jnp.float32(jnp.inf), d2)

    ck = lax.bitcast_convert_type(
        (lax.bitcast_convert_type(d2, jnp.uint32) & jnp.uint32(0xFFFFFC00))
        | lax.bitcast_convert_type(col, jnp.uint32).astype(jnp.uint32),
        jnp.int32)

    imax = jnp.int32(2**31 - 1)
    cols = []
    for _ in range(_K):
        m = jnp.min(ck, axis=1, keepdims=True)
        ck = jnp.where(ck == m, imax, ck)
        cols.append(m & jnp.int32(P - 1))
    r_base = (b * P + rb * R
              + lax.broadcasted_iota(jnp.int32, (R, 1), 0)) * P
    idx_ref[0] = jnp.concatenate(cols, axis=1) + r_base


def _sc_body(pk_hbm, idx_hbm, out_hbm, idx_v, vals_v, acc_v, sem, *,
             n_dmas, n_chunks):
    wid = lax.axis_index("s") * 2 + lax.axis_index("c")
    pltpu.sync_copy(idx_hbm.at[wid], idx_v)
    copies = [
        pltpu.async_copy(pk_hbm.at[idx_v.at[j]],
                         vals_v.at[pl.ds(j * 128, 128)], sem)
        for j in range(n_dmas)
    ]
    for c in copies:
        c.wait()

    hi = jnp.uint32(0xFFFF0000)

    def chunk(i, acc):
        v = vals_v[pl.ds(i * 16, 16)]
        vu = lax.bitcast_convert_type(v, jnp.uint32)
        cs = lax.bitcast_convert_type(vu & hi, jnp.float32)
        ct = lax.bitcast_convert_type(vu << 16, jnp.float32)
        ax = jnp.abs(cs - ct)
        return acc + jnp.where(ax < _BETA, 0.5 * ax * ax / _BETA,
                               ax - 0.5 * _BETA)

    acc_v[...] = lax.fori_loop(0, n_chunks, chunk,
                               jnp.zeros((16,), jnp.float32))
    pltpu.sync_copy(acc_v, out_hbm.at[wid])


def kernel(student_emb, teacher_emb, centers):
    B, P, D = student_emb.shape
    R = min(256, P)
    nrb = P // R

    batch_full = pl.BlockSpec((1, P, D), lambda b: (b, 0, 0))
    s_norm, t_norm = pl.pallas_call(
        _norm_body,
        grid=(B,),
        in_specs=[batch_full, batch_full],
        out_specs=[batch_full, batch_full],
        out_shape=[jax.ShapeDtypeStruct((B, P, D), jnp.bfloat16),
                   jax.ShapeDtypeStruct((B, P, D), jnp.bfloat16)],
        compiler_params=pltpu.CompilerParams(
            dimension_semantics=("arbitrary",)),
    )(student_emb, teacher_emb)

    cblk = pl.BlockSpec((1, R, 3), lambda b, rb: (b, rb if R != P else 0, 0))
    blk = pl.BlockSpec((1, R, D), lambda b, rb: (b, rb if R != P else 0, 0))
    full = pl.BlockSpec((1, P, D), lambda b, rb: (b, 0, 0))
    cfull = pl.BlockSpec((1, P, 3), lambda b, rb: (b, 0, 0))

    pk, idx = pl.pallas_call(
        _tc_body,
        grid=(B, nrb),
        in_specs=[cblk, blk, blk, full, full, cfull],
        out_specs=[pl.BlockSpec((1, R, P), lambda b, rb: (b, rb, 0)),
                   pl.BlockSpec((1, R, _K), lambda b, rb: (b, rb, 0))],
        out_shape=[jax.ShapeDtypeStruct((B, P, P), jnp.int32),
                   jax.ShapeDtypeStruct((B, P, _K), jnp.int32)],
        compiler_params=pltpu.CompilerParams(
            dimension_semantics=("arbitrary", "arbitrary")),
    )(centers, s_norm, t_norm, s_norm, t_norm, centers)

    info = plsc.get_sparse_core_info()
    nw = info.num_cores * info.num_subcores
    n_pairs = B * P * _K
    per_w = n_pairs // nw
    n_dmas = per_w // 128
    n_chunks = per_w // 16

    sc = functools.partial(
        pl.kernel,
        mesh=plsc.VectorSubcoreMesh(core_axis_name="c", subcore_axis_name="s"),
        out_type=jax.ShapeDtypeStruct((nw, 16), jnp.float32),
        scratch_types=[pltpu.VMEM((n_dmas, 128), jnp.int32),
                       pltpu.VMEM((per_w,), jnp.int32),
                       pltpu.VMEM((16,), jnp.float32),
                       pltpu.SemaphoreType.DMA],
    )(functools.partial(_sc_body, n_dmas=n_dmas, n_chunks=n_chunks))

    partials = sc(pk.reshape(-1), idx.reshape(nw, n_dmas, 128))
    return jnp.sum(partials) / jnp.float32(n_pairs)


# speedup vs baseline: 1.2968x; 1.0898x over previous
import functools

import jax
import jax.numpy as jnp
from jax import lax
from jax.experimental import pallas as pl
from jax.experimental.pallas import tpu as pltpu
from jax.experimental.pallas import tpu_sc as plsc

_K = 8
_BETA = 0.5
_EPS = 1e-8


def _tc_body(c_blk_ref, s_full_ref, t_full_ref, c_full_ref, pk_ref, idx_ref,
             s_scr, t_scr):
    b = pl.program_id(0)
    rb = pl.program_id(1)
    R = c_blk_ref.shape[1]
    P = s_full_ref.shape[1]
    D = s_full_ref.shape[2]
    ones_d = jnp.ones((1, D), jnp.float32)
    nt_dims = (((1,), (1,)), ((), ()))
    hi_mask = jnp.uint32(0xFFFF0000)

    @pl.when(rb == 0)
    def _normalize_full():
        for full_ref, scr in ((s_full_ref, s_scr), (t_full_ref, t_scr)):
            x = full_ref[0]
            n2 = lax.dot_general(x * x, ones_d, nt_dims,
                                 preferred_element_type=jnp.float32)
            inv = 1.0 / jnp.maximum(jnp.sqrt(n2), _EPS)
            scr[...] = (x * inv).astype(jnp.bfloat16)

    s_blk = s_scr[pl.ds(rb * R, R), :]
    t_blk = t_scr[pl.ds(rb * R, R), :]
    gs = lax.dot_general(s_blk, s_scr[...], nt_dims,
                         preferred_element_type=jnp.float32)
    gt = lax.dot_general(t_blk, t_scr[...], nt_dims,
                         preferred_element_type=jnp.float32)

    rnd = jnp.uint32(0x8000)
    pk_hi = (lax.bitcast_convert_type(gs, jnp.uint32) + rnd) & hi_mask
    pk_lo = (lax.bitcast_convert_type(gt, jnp.uint32) + rnd) >> 16
    pk_ref[0] = lax.bitcast_convert_type(pk_hi | pk_lo, jnp.int32)

    c_blk = c_blk_ref[0]
    c_full = c_full_ref[0]
    ones_3 = jnp.ones((1, 3), jnp.float32)
    dotc = lax.dot_general(c_blk, c_full, nt_dims,
                           preferred_element_type=jnp.float32)
    n2c_blk = jnp.sum(c_blk * c_blk, axis=1, keepdims=True)
    n2c_full = lax.dot_general(ones_3, c_full * c_full, nt_dims,
                               preferred_element_type=jnp.float32)
    d2 = jnp.maximum(n2c_blk + n2c_full - 2.0 * dotc, 0.0)

    col = lax.broadcasted_iota(jnp.int32, (R, P), 1)
    row = lax.broadcasted_iota(jnp.int32, (R, P), 0) + rb * R
    d2 = jnp.where(col == row, jnp.float32(jnp.inf), d2)

    ck = lax.bitcast_convert_type(
        (lax.bitcast_convert_type(d2, jnp.uint32) & jnp.uint32(0xFFFFFC00))
        | lax.bitcast_convert_type(col, jnp.uint32).astype(jnp.uint32),
        jnp.int32)

    imax = jnp.int32(2**31 - 1)
    cols = []
    for _ in range(_K):
        m = jnp.min(ck, axis=1, keepdims=True)
        ck = jnp.where(ck == m, imax, ck)
        cols.append(m & jnp.int32(P - 1))
    r_base = (b * P + rb * R
              + lax.broadcasted_iota(jnp.int32, (R, 1), 0)) * P
    idx_ref[0] = jnp.concatenate(cols, axis=1) + r_base


def _sc_body(pk_hbm, idx_hbm, out_hbm, idx_v, vals_v, acc_v, sem, *,
             n_dmas, n_chunks):
    wid = lax.axis_index("s") * 2 + lax.axis_index("c")
    pltpu.sync_copy(idx_hbm.at[wid], idx_v)
    copies = [
        pltpu.async_copy(pk_hbm.at[idx_v.at[j]],
                         vals_v.at[pl.ds(j * 128, 128)], sem)
        for j in range(n_dmas)
    ]
    for c in copies:
        c.wait()

    hi = jnp.uint32(0xFFFF0000)

    def chunk(i, acc):
        v = vals_v[pl.ds(i * 16, 16)]
        vu = lax.bitcast_convert_type(v, jnp.uint32)
        cs = lax.bitcast_convert_type(vu & hi, jnp.float32)
        ct = lax.bitcast_convert_type(vu << 16, jnp.float32)
        ax = jnp.abs(cs - ct)
        return acc + jnp.where(ax < _BETA, 0.5 * ax * ax / _BETA,
                               ax - 0.5 * _BETA)

    acc_v[...] = lax.fori_loop(0, n_chunks, chunk,
                               jnp.zeros((16,), jnp.float32))
    pltpu.sync_copy(acc_v, out_hbm.at[wid])


def kernel(student_emb, teacher_emb, centers):
    B, P, D = student_emb.shape
    R = min(256, P)
    nrb = P // R

    cblk = pl.BlockSpec((1, R, 3), lambda b, rb: (b, rb if R != P else 0, 0))
    full = pl.BlockSpec((1, P, D), lambda b, rb: (b, 0, 0))
    cfull = pl.BlockSpec((1, P, 3), lambda b, rb: (b, 0, 0))

    pk, idx = pl.pallas_call(
        _tc_body,
        grid=(B, nrb),
        in_specs=[cblk, full, full, cfull],
        out_specs=[pl.BlockSpec((1, R, P), lambda b, rb: (b, rb, 0)),
                   pl.BlockSpec((1, R, _K), lambda b, rb: (b, rb, 0))],
        out_shape=[jax.ShapeDtypeStruct((B, P, P), jnp.int32),
                   jax.ShapeDtypeStruct((B, P, _K), jnp.int32)],
        scratch_shapes=[pltpu.VMEM((P, D), jnp.bfloat16),
                        pltpu.VMEM((P, D), jnp.bfloat16)],
        compiler_params=pltpu.CompilerParams(
            dimension_semantics=("arbitrary", "arbitrary")),
    )(centers, student_emb, teacher_emb, centers)

    info = plsc.get_sparse_core_info()
    nw = info.num_cores * info.num_subcores
    n_pairs = B * P * _K
    per_w = n_pairs // nw
    n_dmas = per_w // 128
    n_chunks = per_w // 16

    sc = functools.partial(
        pl.kernel,
        mesh=plsc.VectorSubcoreMesh(core_axis_name="c", subcore_axis_name="s"),
        out_type=jax.ShapeDtypeStruct((nw, 16), jnp.float32),
        scratch_types=[pltpu.VMEM((n_dmas, 128), jnp.int32),
                       pltpu.VMEM((per_w,), jnp.int32),
                       pltpu.VMEM((16,), jnp.float32),
                       pltpu.SemaphoreType.DMA],
    )(functools.partial(_sc_body, n_dmas=n_dmas, n_chunks=n_chunks))

    partials = sc(pk.reshape(-1), idx.reshape(nw, n_dmas, 128))
    return jnp.sum(partials) / jnp.float32(n_pairs)


# speedup vs baseline: 1.4608x; 1.1264x over previous
import functools

import jax
import jax.numpy as jnp
from jax import lax
from jax.experimental import pallas as pl
from jax.experimental.pallas import tpu as pltpu
from jax.experimental.pallas import tpu_sc as plsc

_K = 8
_BETA = 0.5
_EPS = 1e-8


def _tc_body(c_blk_ref, s_full_ref, t_full_ref, c_full_ref, pk_ref, idx_ref,
             s_scr, t_scr):
    b = pl.program_id(0)
    rb = pl.program_id(1)
    R = c_blk_ref.shape[1]
    P = s_full_ref.shape[1]
    D = s_full_ref.shape[2]
    ones_d = jnp.ones((1, D), jnp.float32)
    nt_dims = (((1,), (1,)), ((), ()))
    hi_mask = jnp.uint32(0xFFFF0000)

    @pl.when(rb == 0)
    def _normalize_full():
        for full_ref, scr in ((s_full_ref, s_scr), (t_full_ref, t_scr)):
            x = full_ref[0]
            n2 = lax.dot_general(x * x, ones_d, nt_dims,
                                 preferred_element_type=jnp.float32)
            inv = 1.0 / jnp.maximum(jnp.sqrt(n2), _EPS)
            scr[...] = (x * inv).astype(jnp.bfloat16)

    s_blk = s_scr[pl.ds(rb * R, R), :]
    t_blk = t_scr[pl.ds(rb * R, R), :]
    gs = lax.dot_general(s_blk, s_scr[...], nt_dims,
                         preferred_element_type=jnp.float32)
    gt = lax.dot_general(t_blk, t_scr[...], nt_dims,
                         preferred_element_type=jnp.float32)

    rnd = jnp.uint32(0x8000)
    pk_hi = (lax.bitcast_convert_type(gs, jnp.uint32) + rnd) & hi_mask
    pk_lo = (lax.bitcast_convert_type(gt, jnp.uint32) + rnd) >> 16
    pk_ref[...] = lax.bitcast_convert_type(pk_hi | pk_lo, jnp.int32).reshape(-1)

    c_blk = c_blk_ref[0]
    c_full = c_full_ref[0]
    ones_3 = jnp.ones((1, 3), jnp.float32)
    dotc = lax.dot_general(c_blk, c_full, nt_dims,
                           preferred_element_type=jnp.float32)
    n2c_blk = jnp.sum(c_blk * c_blk, axis=1, keepdims=True)
    n2c_full = lax.dot_general(ones_3, c_full * c_full, nt_dims,
                               preferred_element_type=jnp.float32)
    d2 = jnp.maximum(n2c_blk + n2c_full - 2.0 * dotc, 0.0)

    col = lax.broadcasted_iota(jnp.int32, (R, P), 1)
    row = lax.broadcasted_iota(jnp.int32, (R, P), 0) + rb * R
    d2 = jnp.where(col == row, jnp.float32(jnp.inf), d2)

    ck = lax.bitcast_convert_type(
        (lax.bitcast_convert_type(d2, jnp.uint32) & jnp.uint32(0xFFFFFC00))
        | lax.bitcast_convert_type(col, jnp.uint32).astype(jnp.uint32),
        jnp.int32)

    imax = jnp.int32(2**31 - 1)
    cols = []
    for _ in range(_K):
        m = jnp.min(ck, axis=1, keepdims=True)
        ck = jnp.where(ck == m, imax, ck)
        cols.append(m & jnp.int32(P - 1))
    r_base = (b * P + rb * R
              + lax.broadcasted_iota(jnp.int32, (R, 1), 0)) * P
    idx_ref[0] = jnp.concatenate(cols, axis=1) + r_base


def _sc_body(pk_hbm, idx_hbm, out_hbm, idx_v, vals_v, acc_v, sem, *,
             n_dmas, n_chunks):
    wid = lax.axis_index("s") * 2 + lax.axis_index("c")
    pltpu.sync_copy(idx_hbm.at[wid], idx_v)
    copies = [
        pltpu.async_copy(pk_hbm.at[idx_v.at[j]],
                         vals_v.at[pl.ds(j * 128, 128)], sem)
        for j in range(n_dmas)
    ]
    for c in copies:
        c.wait()

    hi = jnp.uint32(0xFFFF0000)

    def chunk(i, acc):
        v = vals_v[pl.ds(i * 16, 16)]
        vu = lax.bitcast_convert_type(v, jnp.uint32)
        cs = lax.bitcast_convert_type(vu & hi, jnp.float32)
        ct = lax.bitcast_convert_type(vu << 16, jnp.float32)
        ax = jnp.abs(cs - ct)
        return acc + jnp.where(ax < _BETA, 0.5 * ax * ax / _BETA,
                               ax - 0.5 * _BETA)

    acc_v[...] = lax.fori_loop(0, n_chunks, chunk,
                               jnp.zeros((16,), jnp.float32))
    pltpu.sync_copy(acc_v, out_hbm.at[wid])


def kernel(student_emb, teacher_emb, centers):
    B, P, D = student_emb.shape
    R = min(256, P)
    nrb = P // R

    cblk = pl.BlockSpec((1, R, 3), lambda b, rb: (b, rb if R != P else 0, 0))
    full = pl.BlockSpec((1, P, D), lambda b, rb: (b, 0, 0))
    cfull = pl.BlockSpec((1, P, 3), lambda b, rb: (b, 0, 0))

    pk, idx = pl.pallas_call(
        _tc_body,
        grid=(B, nrb),
        in_specs=[cblk, full, full, cfull],
        out_specs=[pl.BlockSpec((R * P,), lambda b, rb: (b * nrb + rb,)),
                   pl.BlockSpec((1, R, _K), lambda b, rb: (b, rb, 0))],
        out_shape=[jax.ShapeDtypeStruct((B * P * P,), jnp.int32),
                   jax.ShapeDtypeStruct((B, P, _K), jnp.int32)],
        scratch_shapes=[pltpu.VMEM((P, D), jnp.bfloat16),
                        pltpu.VMEM((P, D), jnp.bfloat16)],
        compiler_params=pltpu.CompilerParams(
            dimension_semantics=("arbitrary", "arbitrary")),
    )(centers, student_emb, teacher_emb, centers)

    info = plsc.get_sparse_core_info()
    nw = info.num_cores * info.num_subcores
    n_pairs = B * P * _K
    per_w = n_pairs // nw
    n_dmas = per_w // 128
    n_chunks = per_w // 16

    sc = functools.partial(
        pl.kernel,
        mesh=plsc.VectorSubcoreMesh(core_axis_name="c", subcore_axis_name="s"),
        out_type=jax.ShapeDtypeStruct((nw, 16), jnp.float32),
        scratch_types=[pltpu.VMEM((n_dmas, 128), jnp.int32),
                       pltpu.VMEM((per_w,), jnp.int32),
                       pltpu.VMEM((16,), jnp.float32),
                       pltpu.SemaphoreType.DMA],
    )(functools.partial(_sc_body, n_dmas=n_dmas, n_chunks=n_chunks))

    partials = sc(pk, idx.reshape(nw, n_dmas, 128))
    return jnp.sum(partials) / jnp.float32(n_pairs)
